# Initial kernel scaffold; baseline (speedup 1.0000x reference)
import jax
import jax.numpy as jnp
from jax.experimental import pallas as pl


def kernel(x, seq_lens, emb, w_ih_l0f, w_hh_l0f, b_ih_l0f, b_hh_l0f, w_ih_l0b, w_hh_l0b, b_ih_l0b, b_hh_l0b, w_ih_l1f, w_hh_l1f, b_ih_l1f, b_hh_l1f, w_ih_l1b, w_hh_l1b, b_ih_l1b, b_hh_l1b, w1, b1, g1, be1, w2, b2, g2, be2, w3, b3, g3, be3, w4, b4):
    raise NotImplementedError("write your pallas kernel here")


# speedup vs baseline: 19.3050x; 19.3050x over previous
import functools

import numpy as np
import jax
import jax.numpy as jnp
from jax.experimental import pallas as pl
from jax.experimental.pallas import tpu as pltpu

H = 256
V = 61
NEG = -1e9

_counts = [4, 6, 2, 2, 2, 2, 2, 4, 2, 3, 6, 1, 1, 2, 4, 6, 4, 4, 1, 2]
_vt = np.zeros((21, V), np.float32)
_c = 0
for _i, _n in enumerate(_counts):
    _vt[_i + 1, _c:_c + _n] = 1.0
    _c += _n
_VALID_CODON = _vt


def _gru_scan_chunk(d, base, CS, lens, gi_scr, whhT, bhh, out_ref, h_scr):
    def step(tt, h):
        t = tt + d * (CS - 1 - 2 * tt)
        gi = gi_scr[t]
        gh = jnp.dot(h, whhT, preferred_element_type=jnp.float32) + bhh
        r = jax.nn.sigmoid(gi[:, :H] + gh[:, :H])
        z = jax.nn.sigmoid(gi[:, H:2 * H] + gh[:, H:2 * H])
        n = jnp.tanh(gi[:, 2 * H:] + r * gh[:, 2 * H:])
        hc = (1.0 - z) * n + z * h
        m = (base + t) < lens
        out_ref[t] = jnp.where(m, hc, 0.0)
        return jnp.where(m, hc, h)

    h_scr[...] = jax.lax.fori_loop(0, CS, step, h_scr[...])


def _l0_body(x_ref, lens_ref, emb_ref, wih_ref, bih_ref, whhT_ref, bhh_ref,
             out_ref, h_scr, gi_scr, *, CS, NC):
    d = pl.program_id(0)
    i = pl.program_id(1)
    chunk = i + d * (NC - 1 - 2 * i)
    base = chunk * CS

    @pl.when(i == 0)
    def _():
        h_scr[...] = jnp.zeros_like(h_scr)

    table = jax.lax.dot_general(
        emb_ref[...], wih_ref[0], (((1,), (1,)), ((), ())),
        preferred_element_type=jnp.float32) + bih_ref[0]

    B = x_ref.shape[1]
    for sl in range(CS // 32):
        xs = x_ref[sl * 32:(sl + 1) * 32]
        oh = (xs[:, :, None] ==
              jax.lax.broadcasted_iota(jnp.int32, (32, B, 21), 2)
              ).astype(jnp.float32)
        gi = jnp.dot(oh.reshape(32 * B, 21), table,
                     preferred_element_type=jnp.float32)
        gi_scr[sl * 32:(sl + 1) * 32] = gi.reshape(32, B, 3 * H)

    _gru_scan_chunk(d, base, CS, lens_ref[...], gi_scr, whhT_ref[0],
                    bhh_ref[0], out_ref, h_scr)


def _l1_body(hf_ref, hb_ref, lens_ref, wiT_ref, bih_ref, whhT_ref, bhh_ref,
             out_ref, h_scr, gi_scr, *, CS, NC):
    d = pl.program_id(0)
    i = pl.program_id(1)
    chunk = i + d * (NC - 1 - 2 * i)
    base = chunk * CS

    @pl.when(i == 0)
    def _():
        h_scr[...] = jnp.zeros_like(h_scr)

    B = hf_ref.shape[1]
    wf = wiT_ref[0, :H]
    wb = wiT_ref[0, H:]
    bih = bih_ref[0]
    for sl in range(CS // 16):
        hf2 = hf_ref[sl * 16:(sl + 1) * 16].reshape(16 * B, H)
        hb2 = hb_ref[sl * 16:(sl + 1) * 16].reshape(16 * B, H)
        gi = (jnp.dot(hf2, wf, preferred_element_type=jnp.float32) +
              jnp.dot(hb2, wb, preferred_element_type=jnp.float32) + bih)
        gi_scr[sl * 16:(sl + 1) * 16] = gi.reshape(16, B, 3 * H)

    _gru_scan_chunk(d, base, CS, lens_ref[...], gi_scr, whhT_ref[0],
                    bhh_ref[0], out_ref, h_scr)


def _head1_body(hf_ref, hb_ref, w1T_ref, b1_ref, o_ref, ps_ref):
    rows = hf_ref.shape[0] * hf_ref.shape[1]
    hf2 = hf_ref[...].reshape(rows, H)
    hb2 = hb_ref[...].reshape(rows, H)
    o = (jnp.dot(hf2, w1T_ref[:H], preferred_element_type=jnp.float32) +
         jnp.dot(hb2, w1T_ref[H:], preferred_element_type=jnp.float32) +
         b1_ref[...])
    o_ref[...] = o
    ps_ref[0] = jnp.concatenate(
        [jnp.sum(o, 0)[None], jnp.sum(o * o, 0)[None]], 0)


def _bn_leaky(o, ps, g, be, ntok):
    tot = jnp.sum(ps, axis=0)
    mean = tot[0:1] / ntok
    var = tot[1:2] / ntok - mean * mean
    a = g * (o - mean) * jax.lax.rsqrt(var + 1e-5) + be
    return jnp.where(a >= 0, a, 0.01 * a)


def _head_mid_body(o_in_ref, ps_in_ref, g_ref, be_ref, wT_ref, b_ref,
                   o_ref, ps_ref, *, ntok):
    a = _bn_leaky(o_in_ref[...], ps_in_ref[...], g_ref[...], be_ref[...],
                  ntok)
    o = jnp.dot(a, wT_ref[...], preferred_element_type=jnp.float32) + b_ref[...]
    o_ref[...] = o
    ps_ref[0] = jnp.concatenate(
        [jnp.sum(o, 0)[None], jnp.sum(o * o, 0)[None]], 0)


def _head4_body(o_in_ref, ps_in_ref, g_ref, be_ref, wT_ref, b_ref, x_ref,
                vt_ref, out_ref, *, ntok):
    a = _bn_leaky(o_in_ref[...], ps_in_ref[...], g_ref[...], be_ref[...],
                  ntok)
    o = jnp.dot(a, wT_ref[...], preferred_element_type=jnp.float32) + b_ref[...]
    mx = jnp.max(o, axis=1, keepdims=True)
    e = jnp.exp(o - mx)
    p = e / jnp.sum(e, axis=1, keepdims=True)
    CSX, B = x_ref.shape
    oh = (x_ref[...][:, :, None] ==
          jax.lax.broadcasted_iota(jnp.int32, (CSX, B, 21), 2)
          ).astype(jnp.float32)
    vm = jnp.dot(oh.reshape(CSX * B, 21), vt_ref[...],
                 preferred_element_type=jnp.float32)
    out_ref[...] = p + (vm - 1.0) * (-NEG)


def kernel(x, seq_lens, emb,
           w_ih_l0f, w_hh_l0f, b_ih_l0f, b_hh_l0f,
           w_ih_l0b, w_hh_l0b, b_ih_l0b, b_hh_l0b,
           w_ih_l1f, w_hh_l1f, b_ih_l1f, b_hh_l1f,
           w_ih_l1b, w_hh_l1b, b_ih_l1b, b_hh_l1b,
           w1, b1, g1, be1, w2, b2, g2, be2,
           w3, b3, g3, be3, w4, b4):
    B, S = x.shape
    E = emb.shape[1]
    T = S * B
    ntok = float(T)

    CS0 = min(128, S)
    NC0 = S // CS0
    CS1 = min(64, S)
    NC1 = S // CS1
    TCS = min(128, S)
    NT = S // TCS
    TB = TCS * B

    xT = x.T
    lens2 = seq_lens.reshape(B, 1)
    cp = functools.partial(pltpu.CompilerParams,
                           vmem_limit_bytes=56 * 1024 * 1024)

    wih0 = jnp.stack([w_ih_l0f, w_ih_l0b])
    bih0 = jnp.stack([b_ih_l0f, b_ih_l0b])[:, None]
    whhT0 = jnp.stack([w_hh_l0f.T, w_hh_l0b.T])
    bhh0 = jnp.stack([b_hh_l0f, b_hh_l0b])[:, None]

    h0 = pl.pallas_call(
        functools.partial(_l0_body, CS=CS0, NC=NC0),
        grid=(2, NC0),
        in_specs=[
            pl.BlockSpec((CS0, B), lambda d, i: (i + d * (NC0 - 1 - 2 * i), 0)),
            pl.BlockSpec((B, 1), lambda d, i: (0, 0)),
            pl.BlockSpec((21, E), lambda d, i: (0, 0)),
            pl.BlockSpec((1, 3 * H, E), lambda d, i: (d, 0, 0)),
            pl.BlockSpec((1, 1, 3 * H), lambda d, i: (d, 0, 0)),
            pl.BlockSpec((1, H, 3 * H), lambda d, i: (d, 0, 0)),
            pl.BlockSpec((1, 1, 3 * H), lambda d, i: (d, 0, 0)),
        ],
        out_specs=pl.BlockSpec(
            (CS0, B, H),
            lambda d, i: (d * NC0 + i + d * (NC0 - 1 - 2 * i), 0, 0)),
        out_shape=jax.ShapeDtypeStruct((2 * S, B, H), jnp.float32),
        scratch_shapes=[pltpu.VMEM((B, H), jnp.float32),
                        pltpu.VMEM((CS0, B, 3 * H), jnp.float32)],
        compiler_params=cp(dimension_semantics=("parallel", "arbitrary")),
        name="bigru_l0",
    )(xT, lens2, emb, wih0, bih0, whhT0, bhh0)

    wiT1 = jnp.stack([w_ih_l1f.T, w_ih_l1b.T])
    bih1 = jnp.stack([b_ih_l1f, b_ih_l1b])[:, None]
    whhT1 = jnp.stack([w_hh_l1f.T, w_hh_l1b.T])
    bhh1 = jnp.stack([b_hh_l1f, b_hh_l1b])[:, None]

    h1 = pl.pallas_call(
        functools.partial(_l1_body, CS=CS1, NC=NC1),
        grid=(2, NC1),
        in_specs=[
            pl.BlockSpec((CS1, B, H),
                         lambda d, i: (i + d * (NC1 - 1 - 2 * i), 0, 0)),
            pl.BlockSpec((CS1, B, H),
                         lambda d, i: (NC1 + i + d * (NC1 - 1 - 2 * i), 0, 0)),
            pl.BlockSpec((B, 1), lambda d, i: (0, 0)),
            pl.BlockSpec((1, 2 * H, 3 * H), lambda d, i: (d, 0, 0)),
            pl.BlockSpec((1, 1, 3 * H), lambda d, i: (d, 0, 0)),
            pl.BlockSpec((1, H, 3 * H), lambda d, i: (d, 0, 0)),
            pl.BlockSpec((1, 1, 3 * H), lambda d, i: (d, 0, 0)),
        ],
        out_specs=pl.BlockSpec(
            (CS1, B, H),
            lambda d, i: (d * NC1 + i + d * (NC1 - 1 - 2 * i), 0, 0)),
        out_shape=jax.ShapeDtypeStruct((2 * S, B, H), jnp.float32),
        scratch_shapes=[pltpu.VMEM((B, H), jnp.float32),
                        pltpu.VMEM((CS1, B, 3 * H), jnp.float32)],
        compiler_params=cp(dimension_semantics=("parallel", "arbitrary")),
        name="bigru_l1",
    )(h0, h0, lens2, wiT1, bih1, whhT1, bhh1)

    o1, ps1 = pl.pallas_call(
        _head1_body,
        grid=(NT,),
        in_specs=[
            pl.BlockSpec((TCS, B, H), lambda i: (i, 0, 0)),
            pl.BlockSpec((TCS, B, H), lambda i: (NT + i, 0, 0)),
            pl.BlockSpec((2 * H, 128), lambda i: (0, 0)),
            pl.BlockSpec((1, 128), lambda i: (0, 0)),
        ],
        out_specs=[
            pl.BlockSpec((TB, 128), lambda i: (i, 0)),
            pl.BlockSpec((1, 2, 128), lambda i: (i, 0, 0)),
        ],
        out_shape=[jax.ShapeDtypeStruct((T, 128), jnp.float32),
                   jax.ShapeDtypeStruct((NT, 2, 128), jnp.float32)],
        compiler_params=cp(dimension_semantics=("parallel",)),
        name="head_l1",
    )(h1, h1, w1.T, b1[None])

    o2, ps2 = pl.pallas_call(
        functools.partial(_head_mid_body, ntok=ntok),
        grid=(NT,),
        in_specs=[
            pl.BlockSpec((TB, 128), lambda i: (i, 0)),
            pl.BlockSpec((NT, 2, 128), lambda i: (0, 0, 0)),
            pl.BlockSpec((1, 128), lambda i: (0, 0)),
            pl.BlockSpec((1, 128), lambda i: (0, 0)),
            pl.BlockSpec((128, 256), lambda i: (0, 0)),
            pl.BlockSpec((1, 256), lambda i: (0, 0)),
        ],
        out_specs=[
            pl.BlockSpec((TB, 256), lambda i: (i, 0)),
            pl.BlockSpec((1, 2, 256), lambda i: (i, 0, 0)),
        ],
        out_shape=[jax.ShapeDtypeStruct((T, 256), jnp.float32),
                   jax.ShapeDtypeStruct((NT, 2, 256), jnp.float32)],
        compiler_params=cp(dimension_semantics=("parallel",)),
        name="head_l2",
    )(o1, ps1, g1[None], be1[None], w2.T, b2[None])

    o3, ps3 = pl.pallas_call(
        functools.partial(_head_mid_body, ntok=ntok),
        grid=(NT,),
        in_specs=[
            pl.BlockSpec((TB, 256), lambda i: (i, 0)),
            pl.BlockSpec((NT, 2, 256), lambda i: (0, 0, 0)),
            pl.BlockSpec((1, 256), lambda i: (0, 0)),
            pl.BlockSpec((1, 256), lambda i: (0, 0)),
            pl.BlockSpec((256, 128), lambda i: (0, 0)),
            pl.BlockSpec((1, 128), lambda i: (0, 0)),
        ],
        out_specs=[
            pl.BlockSpec((TB, 128), lambda i: (i, 0)),
            pl.BlockSpec((1, 2, 128), lambda i: (i, 0, 0)),
        ],
        out_shape=[jax.ShapeDtypeStruct((T, 128), jnp.float32),
                   jax.ShapeDtypeStruct((NT, 2, 128), jnp.float32)],
        compiler_params=cp(dimension_semantics=("parallel",)),
        name="head_l3",
    )(o2, ps2, g2[None], be2[None], w3.T, b3[None])

    vt = jnp.asarray(_VALID_CODON)
    out = pl.pallas_call(
        functools.partial(_head4_body, ntok=ntok),
        grid=(NT,),
        in_specs=[
            pl.BlockSpec((TB, 128), lambda i: (i, 0)),
            pl.BlockSpec((NT, 2, 128), lambda i: (0, 0, 0)),
            pl.BlockSpec((1, 128), lambda i: (0, 0)),
            pl.BlockSpec((1, 128), lambda i: (0, 0)),
            pl.BlockSpec((128, V), lambda i: (0, 0)),
            pl.BlockSpec((1, V), lambda i: (0, 0)),
            pl.BlockSpec((TCS, B), lambda i: (i, 0)),
            pl.BlockSpec((21, V), lambda i: (0, 0)),
        ],
        out_specs=pl.BlockSpec((TB, V), lambda i: (i, 0)),
        out_shape=jax.ShapeDtypeStruct((T, V), jnp.float32),
        compiler_params=cp(dimension_semantics=("parallel",)),
        name="head_l4",
    )(o3, ps3, g3[None], be3[None], w4.T, b4[None], xT, vt)

    return out.reshape(S, B, V).swapaxes(0, 1)


# speedup vs baseline: 26.5410x; 1.3748x over previous
import functools

import numpy as np
import jax
import jax.numpy as jnp
from jax.experimental import pallas as pl
from jax.experimental.pallas import tpu as pltpu

H = 256
V = 61
NEG = -1e9
BF = jnp.bfloat16
F32 = jnp.float32

_counts = [4, 6, 2, 2, 2, 2, 2, 4, 2, 3, 6, 1, 1, 2, 4, 6, 4, 4, 1, 2]
_vt = np.zeros((21, V), np.float32)
_c = 0
for _i, _n in enumerate(_counts):
    _vt[_i + 1, _c:_c + _n] = 1.0
    _c += _n
_VALID_CODON = _vt


def _gates(gi, gh, h, bhn):
    r = jax.nn.sigmoid(gi[:, :H] + gh[:, :H])
    z = jax.nn.sigmoid(gi[:, H:2 * H] + gh[:, H:2 * H])
    n = jnp.tanh(gi[:, 2 * H:] + r * (gh[:, 2 * H:] + bhn))
    return (1.0 - z) * n + z * h


def _scan_dual(CS, base_f, base_b, lens, gif_scr, gib_scr, whh_ref,
               bhnf, bhnb, outf_ref, outb_ref, hf_scr, hb_scr):
    B = hf_scr.shape[0]
    wf = whh_ref.at[0]
    wb = whh_ref.at[1]
    pltpu.matmul_push_rhs(wf[:, 0 * 256:1 * 256], 0, 0)
    pltpu.matmul_push_rhs(wf[:, 1 * 256:2 * 256], 1, 0)
    pltpu.matmul_push_rhs(wb[:, 0 * 256:1 * 256], 0, 1)
    pltpu.matmul_push_rhs(wb[:, 1 * 256:2 * 256], 1, 1)

    def step(tt, carry):
        hf, hb = carry
        tb = CS - 1 - tt
        gif = gif_scr[tt]
        gib = gib_scr[tb]
        lf = hf.astype(BF)
        lb = hb.astype(BF)
        pltpu.matmul_acc_lhs(0, lf, 0, load_staged_rhs=0)
        pltpu.matmul_acc_lhs(16, lf, 0, load_staged_rhs=1)
        pltpu.matmul_push_rhs(wf[:, 2 * 256:3 * 256], 1, 0)
        pltpu.matmul_acc_lhs(32, lf, 0, load_staged_rhs=1)
        pltpu.matmul_push_rhs(wf[:, 1 * 256:2 * 256], 1, 0)
        pltpu.matmul_acc_lhs(0, lb, 1, load_staged_rhs=0)
        pltpu.matmul_acc_lhs(16, lb, 1, load_staged_rhs=1)
        pltpu.matmul_push_rhs(wb[:, 2 * 256:3 * 256], 1, 1)
        pltpu.matmul_acc_lhs(32, lb, 1, load_staged_rhs=1)
        pltpu.matmul_push_rhs(wb[:, 1 * 256:2 * 256], 1, 1)
        ghf = jnp.concatenate(
            [pltpu.matmul_pop(0, (B, 256), F32, 0),
             pltpu.matmul_pop(16, (B, 256), F32, 0),
             pltpu.matmul_pop(32, (B, 256), F32, 0)], axis=1)
        ghb = jnp.concatenate(
            [pltpu.matmul_pop(0, (B, 256), F32, 1),
             pltpu.matmul_pop(16, (B, 256), F32, 1),
             pltpu.matmul_pop(32, (B, 256), F32, 1)], axis=1)
        hcf = _gates(gif, ghf, hf, bhnf)
        hcb = _gates(gib, ghb, hb, bhnb)
        mf = (base_f + tt) < lens
        mb = (base_b + tb) < lens
        outf_ref[tt] = jnp.where(mf, hcf, 0.0).astype(BF)
        outb_ref[tb] = jnp.where(mb, hcb, 0.0).astype(BF)
        return jnp.where(mf, hcf, hf), jnp.where(mb, hcb, hb)

    hf, hb = jax.lax.fori_loop(0, CS, step, (hf_scr[...], hb_scr[...]))
    hf_scr[...] = hf
    hb_scr[...] = hb
    drain = jnp.zeros((16, 256), BF)
    pltpu.matmul_acc_lhs(48, drain, 0, load_staged_rhs=1)
    pltpu.matmul_pop(48, (16, 256), F32, 0)
    pltpu.matmul_acc_lhs(48, drain, 1, load_staged_rhs=1)
    pltpu.matmul_pop(48, (16, 256), F32, 1)


def _onehot_gi(x_ref, tbl, gi_scr, CS, B):
    rows = 4 * B
    pltpu.matmul_push_rhs(tbl[0], 0, 0)
    pltpu.matmul_push_rhs(tbl[1], 1, 0)
    pltpu.matmul_push_rhs(tbl[2], 0, 1)
    for sl in range(CS // 4):
        xs = x_ref[sl * 4:(sl + 1) * 4]
        oh = (xs[:, :, None] ==
              jax.lax.broadcasted_iota(jnp.int32, (4, B, 256), 2)
              ).astype(BF).reshape(rows, 256)
        pltpu.matmul_acc_lhs(0, oh, 0, load_staged_rhs=0)
        pltpu.matmul_acc_lhs(64, oh, 0, load_staged_rhs=1)
        pltpu.matmul_acc_lhs(0, oh, 1, load_staged_rhs=0)
        gi = jnp.concatenate(
            [pltpu.matmul_pop(0, (rows, 256), F32, 0),
             pltpu.matmul_pop(64, (rows, 256), F32, 0),
             pltpu.matmul_pop(0, (rows, 256), F32, 1)], axis=1)
        gi_scr[sl * 4:(sl + 1) * 4] = gi.reshape(4, B, 3 * H).astype(BF)


def _l0_body(xf_ref, xb_ref, lens_ref, tbl_ref, whhT_ref, bhh_ref,
             outf_ref, outb_ref, hf_scr, hb_scr, gif_scr, gib_scr,
             *, CS, NC):
    i = pl.program_id(0)
    base_f = i * CS
    base_b = (NC - 1 - i) * CS

    @pl.when(i == 0)
    def _():
        hf_scr[...] = jnp.zeros_like(hf_scr)
        hb_scr[...] = jnp.zeros_like(hb_scr)

    B = xf_ref.shape[1]
    _onehot_gi(xf_ref, tbl_ref.at[0], gif_scr, CS, B)
    _onehot_gi(xb_ref, tbl_ref.at[1], gib_scr, CS, B)
    _scan_dual(CS, base_f, base_b, lens_ref[...], gif_scr, gib_scr,
               whhT_ref, bhh_ref[0][:, 2 * H:], bhh_ref[1][:, 2 * H:],
               outf_ref, outb_ref, hf_scr, hb_scr)


def _lin_gi(hf_ref, hb_ref, wiT, bih, gi_scr, CS, B):
    rows = 4 * B
    pltpu.matmul_push_rhs(wiT[0 * 256:1 * 256, 0 * 256:1 * 256], 0, 0)
    pltpu.matmul_push_rhs(wiT[1 * 256:2 * 256, 0 * 256:1 * 256], 1, 0)
    pltpu.matmul_push_rhs(wiT[0 * 256:1 * 256, 1 * 256:2 * 256], 0, 1)
    pltpu.matmul_push_rhs(wiT[1 * 256:2 * 256, 1 * 256:2 * 256], 1, 1)
    for sl in range(CS // 4):
        hf2 = hf_ref[sl * 4:(sl + 1) * 4].reshape(rows, H)
        hb2 = hb_ref[sl * 4:(sl + 1) * 4].reshape(rows, H)
        pltpu.matmul_acc_lhs(0, hf2, 0, load_staged_rhs=0)
        pltpu.matmul_acc_lhs(0, hf2, 1, load_staged_rhs=0)
        pltpu.matmul_acc_lhs(0, hb2, 0, load_staged_rhs=1)
        pltpu.matmul_acc_lhs(0, hb2, 1, load_staged_rhs=1)
        g0 = pltpu.matmul_pop(0, (rows, 256), F32, 0) + bih[:, :256]
        g1 = pltpu.matmul_pop(0, (rows, 256), F32, 1) + bih[:, 256:512]
        gi_scr[sl * 4:(sl + 1) * 4, :, 0:256] = (
            g0.reshape(4, B, 256).astype(BF))
        gi_scr[sl * 4:(sl + 1) * 4, :, 256:512] = (
            g1.reshape(4, B, 256).astype(BF))
    pltpu.matmul_push_rhs(wiT[0 * 256:1 * 256, 2 * 256:3 * 256], 0, 0)
    pltpu.matmul_push_rhs(wiT[1 * 256:2 * 256, 2 * 256:3 * 256], 0, 1)
    for sl in range(CS // 4):
        hf2 = hf_ref[sl * 4:(sl + 1) * 4].reshape(rows, H)
        hb2 = hb_ref[sl * 4:(sl + 1) * 4].reshape(rows, H)
        pltpu.matmul_acc_lhs(0, hf2, 0, load_staged_rhs=0)
        pltpu.matmul_acc_lhs(0, hb2, 1, load_staged_rhs=0)
        g2 = (pltpu.matmul_pop(0, (rows, 256), F32, 0) +
              pltpu.matmul_pop(0, (rows, 256), F32, 1) + bih[:, 512:])
        gi_scr[sl * 4:(sl + 1) * 4, :, 512:768] = (
            g2.reshape(4, B, 256).astype(BF))


def _l1_body(hff_ref, hfb_ref, hbf_ref, hbb_ref, lens_ref, wiT_ref,
             bih_ref, whhT_ref, bhh_ref, outf_ref, outb_ref,
             hf_scr, hb_scr, gif_scr, gib_scr, *, CS, NC):
    i = pl.program_id(0)
    base_f = i * CS
    base_b = (NC - 1 - i) * CS

    @pl.when(i == 0)
    def _():
        hf_scr[...] = jnp.zeros_like(hf_scr)
        hb_scr[...] = jnp.zeros_like(hb_scr)

    B = hff_ref.shape[1]
    _lin_gi(hff_ref, hbf_ref, wiT_ref.at[0], bih_ref[0], gif_scr, CS, B)
    _lin_gi(hfb_ref, hbb_ref, wiT_ref.at[1], bih_ref[1], gib_scr, CS, B)
    _scan_dual(CS, base_f, base_b, lens_ref[...], gif_scr, gib_scr,
               whhT_ref, bhh_ref[0][:, 2 * H:], bhh_ref[1][:, 2 * H:],
               outf_ref, outb_ref, hf_scr, hb_scr)


def _tbl_body(emb_ref, wih_ref, bih_ref, out_ref):
    t = jax.lax.dot_general(
        emb_ref[...], wih_ref[0], (((1,), (1,)), ((), ())),
        preferred_element_type=F32) + bih_ref[0]
    t = t.astype(BF)
    out_ref[0, 0] = t[:, 0 * 256:1 * 256]
    out_ref[0, 1] = t[:, 1 * 256:2 * 256]
    out_ref[0, 2] = t[:, 2 * 256:3 * 256]


def _head1_body(hf_ref, hb_ref, w1T_ref, b1_ref, o_ref, ps_ref):
    rows = hf_ref.shape[0] * hf_ref.shape[1]
    hf2 = hf_ref[...].reshape(rows, H)
    hb2 = hb_ref[...].reshape(rows, H)
    o = (jnp.dot(hf2, w1T_ref[:H], preferred_element_type=F32) +
         jnp.dot(hb2, w1T_ref[H:], preferred_element_type=F32) +
         b1_ref[...])
    o_ref[...] = o.astype(BF)
    ps_ref[0] = jnp.concatenate(
        [jnp.sum(o, 0)[None], jnp.sum(o * o, 0)[None]], 0)


def _bn_leaky(o, ps, g, be, ntok):
    tot = jnp.sum(ps, axis=0)
    mean = tot[0:1] / ntok
    var = tot[1:2] / ntok - mean * mean
    a = g * (o - mean) * jax.lax.rsqrt(var + 1e-5) + be
    return jnp.where(a >= 0, a, 0.01 * a).astype(BF)


def _head_mid_body(o_in_ref, ps_in_ref, g_ref, be_ref, wT_ref, b_ref,
                   o_ref, ps_ref, *, ntok):
    a = _bn_leaky(o_in_ref[...].astype(F32), ps_in_ref[...],
                  g_ref[...], be_ref[...], ntok)
    o = jnp.dot(a, wT_ref[...], preferred_element_type=F32) + b_ref[...]
    o_ref[...] = o.astype(BF)
    ps_ref[0] = jnp.concatenate(
        [jnp.sum(o, 0)[None], jnp.sum(o * o, 0)[None]], 0)


def _head4_body(o_in_ref, ps_in_ref, g_ref, be_ref, wT_ref, b_ref, x_ref,
                vt_ref, out_ref, *, ntok):
    a = _bn_leaky(o_in_ref[...].astype(F32), ps_in_ref[...],
                  g_ref[...], be_ref[...], ntok)
    o = jnp.dot(a, wT_ref[...], preferred_element_type=F32) + b_ref[...]
    mx = jnp.max(o, axis=1, keepdims=True)
    e = jnp.exp(o - mx)
    p = e / jnp.sum(e, axis=1, keepdims=True)
    CSX, B = x_ref.shape
    oh = (x_ref[...][:, :, None] ==
          jax.lax.broadcasted_iota(jnp.int32, (CSX, B, 21), 2)
          ).astype(F32)
    vm = jnp.dot(oh.reshape(CSX * B, 21), vt_ref[...],
                 preferred_element_type=F32)
    out_ref[...] = p + (vm - 1.0) * (-NEG)


def kernel(x, seq_lens, emb,
           w_ih_l0f, w_hh_l0f, b_ih_l0f, b_hh_l0f,
           w_ih_l0b, w_hh_l0b, b_ih_l0b, b_hh_l0b,
           w_ih_l1f, w_hh_l1f, b_ih_l1f, b_hh_l1f,
           w_ih_l1b, w_hh_l1b, b_ih_l1b, b_hh_l1b,
           w1, b1, g1, be1, w2, b2, g2, be2,
           w3, b3, g3, be3, w4, b4):
    B, S = x.shape
    E = emb.shape[1]
    T = S * B
    ntok = float(T)

    CS0 = min(128, S)
    NC0 = S // CS0
    CS1 = min(64, S)
    NC1 = S // CS1
    TCS = min(128, S)
    NT = S // TCS
    TB = TCS * B

    xT = x.T
    lens2 = seq_lens.reshape(B, 1)
    cp = functools.partial(pltpu.CompilerParams,
                           vmem_limit_bytes=56 * 1024 * 1024)

    emb_pad = jnp.zeros((256, E), F32).at[:21].set(emb)
    wih0 = jnp.stack([w_ih_l0f, w_ih_l0b])
    _rz = lambda v: jnp.concatenate(
        [v[..., :2 * H], jnp.zeros_like(v[..., 2 * H:])], axis=-1)
    bhh0 = jnp.stack([b_hh_l0f, b_hh_l0b])[:, None]
    bih0 = jnp.stack([b_ih_l0f, b_ih_l0b])[:, None] + _rz(bhh0)
    tbl0 = pl.pallas_call(
        _tbl_body,
        grid=(2,),
        in_specs=[
            pl.BlockSpec((256, E), lambda d: (0, 0)),
            pl.BlockSpec((1, 3 * H, E), lambda d: (d, 0, 0)),
            pl.BlockSpec((1, 1, 3 * H), lambda d: (d, 0, 0)),
        ],
        out_specs=pl.BlockSpec((1, 3, 256, 256), lambda d: (d, 0, 0, 0)),
        out_shape=jax.ShapeDtypeStruct((2, 3, 256, 256), BF),
        name="gate_tables",
    )(emb_pad, wih0, bih0)

    whhT0 = jnp.stack([w_hh_l0f.T, w_hh_l0b.T]).astype(BF)

    h0f, h0b = pl.pallas_call(
        functools.partial(_l0_body, CS=CS0, NC=NC0),
        grid=(NC0,),
        in_specs=[
            pl.BlockSpec((CS0, B), lambda i: (i, 0)),
            pl.BlockSpec((CS0, B), lambda i: (NC0 - 1 - i, 0)),
            pl.BlockSpec((B, 1), lambda i: (0, 0)),
            pl.BlockSpec((2, 3, 256, 256), lambda i: (0, 0, 0, 0)),
            pl.BlockSpec((2, H, 3 * H), lambda i: (0, 0, 0)),
            pl.BlockSpec((2, 1, 3 * H), lambda i: (0, 0, 0)),
        ],
        out_specs=[
            pl.BlockSpec((CS0, B, H), lambda i: (i, 0, 0)),
            pl.BlockSpec((CS0, B, H), lambda i: (NC0 - 1 - i, 0, 0)),
        ],
        out_shape=[jax.ShapeDtypeStruct((S, B, H), BF),
                   jax.ShapeDtypeStruct((S, B, H), BF)],
        scratch_shapes=[pltpu.VMEM((B, H), F32),
                        pltpu.VMEM((B, H), F32),
                        pltpu.VMEM((CS0, B, 3 * H), BF),
                        pltpu.VMEM((CS0, B, 3 * H), BF)],
        compiler_params=cp(dimension_semantics=("arbitrary",)),
        name="bigru_l0",
    )(xT, xT, lens2, tbl0, whhT0, bhh0)

    wiT1 = jnp.stack([w_ih_l1f.T, w_ih_l1b.T]).astype(BF)
    bhh1 = jnp.stack([b_hh_l1f, b_hh_l1b])[:, None]
    bih1 = jnp.stack([b_ih_l1f, b_ih_l1b])[:, None] + _rz(bhh1)
    whhT1 = jnp.stack([w_hh_l1f.T, w_hh_l1b.T]).astype(BF)

    h1f, h1b = pl.pallas_call(
        functools.partial(_l1_body, CS=CS1, NC=NC1),
        grid=(NC1,),
        in_specs=[
            pl.BlockSpec((CS1, B, H), lambda i: (i, 0, 0)),
            pl.BlockSpec((CS1, B, H), lambda i: (NC1 - 1 - i, 0, 0)),
            pl.BlockSpec((CS1, B, H), lambda i: (i, 0, 0)),
            pl.BlockSpec((CS1, B, H), lambda i: (NC1 - 1 - i, 0, 0)),
            pl.BlockSpec((B, 1), lambda i: (0, 0)),
            pl.BlockSpec((2, 2 * H, 3 * H), lambda i: (0, 0, 0)),
            pl.BlockSpec((2, 1, 3 * H), lambda i: (0, 0, 0)),
            pl.BlockSpec((2, H, 3 * H), lambda i: (0, 0, 0)),
            pl.BlockSpec((2, 1, 3 * H), lambda i: (0, 0, 0)),
        ],
        out_specs=[
            pl.BlockSpec((CS1, B, H), lambda i: (i, 0, 0)),
            pl.BlockSpec((CS1, B, H), lambda i: (NC1 - 1 - i, 0, 0)),
        ],
        out_shape=[jax.ShapeDtypeStruct((S, B, H), BF),
                   jax.ShapeDtypeStruct((S, B, H), BF)],
        scratch_shapes=[pltpu.VMEM((B, H), F32),
                        pltpu.VMEM((B, H), F32),
                        pltpu.VMEM((CS1, B, 3 * H), BF),
                        pltpu.VMEM((CS1, B, 3 * H), BF)],
        compiler_params=cp(dimension_semantics=("arbitrary",)),
        name="bigru_l1",
    )(h0f, h0f, h0b, h0b, lens2, wiT1, bih1, whhT1, bhh1)

    o1, ps1 = pl.pallas_call(
        _head1_body,
        grid=(NT,),
        in_specs=[
            pl.BlockSpec((TCS, B, H), lambda i: (i, 0, 0)),
            pl.BlockSpec((TCS, B, H), lambda i: (i, 0, 0)),
            pl.BlockSpec((2 * H, 128), lambda i: (0, 0)),
            pl.BlockSpec((1, 128), lambda i: (0, 0)),
        ],
        out_specs=[
            pl.BlockSpec((TB, 128), lambda i: (i, 0)),
            pl.BlockSpec((1, 2, 128), lambda i: (i, 0, 0)),
        ],
        out_shape=[jax.ShapeDtypeStruct((T, 128), BF),
                   jax.ShapeDtypeStruct((NT, 2, 128), F32)],
        compiler_params=cp(dimension_semantics=("parallel",)),
        name="head_l1",
    )(h1f, h1b, w1.T.astype(BF), b1[None])

    o2, ps2 = pl.pallas_call(
        functools.partial(_head_mid_body, ntok=ntok),
        grid=(NT,),
        in_specs=[
            pl.BlockSpec((TB, 128), lambda i: (i, 0)),
            pl.BlockSpec((NT, 2, 128), lambda i: (0, 0, 0)),
            pl.BlockSpec((1, 128), lambda i: (0, 0)),
            pl.BlockSpec((1, 128), lambda i: (0, 0)),
            pl.BlockSpec((128, 256), lambda i: (0, 0)),
            pl.BlockSpec((1, 256), lambda i: (0, 0)),
        ],
        out_specs=[
            pl.BlockSpec((TB, 256), lambda i: (i, 0)),
            pl.BlockSpec((1, 2, 256), lambda i: (i, 0, 0)),
        ],
        out_shape=[jax.ShapeDtypeStruct((T, 256), BF),
                   jax.ShapeDtypeStruct((NT, 2, 256), F32)],
        compiler_params=cp(dimension_semantics=("parallel",)),
        name="head_l2",
    )(o1, ps1, g1[None], be1[None], w2.T.astype(BF), b2[None])

    o3, ps3 = pl.pallas_call(
        functools.partial(_head_mid_body, ntok=ntok),
        grid=(NT,),
        in_specs=[
            pl.BlockSpec((TB, 256), lambda i: (i, 0)),
            pl.BlockSpec((NT, 2, 256), lambda i: (0, 0, 0)),
            pl.BlockSpec((1, 256), lambda i: (0, 0)),
            pl.BlockSpec((1, 256), lambda i: (0, 0)),
            pl.BlockSpec((256, 128), lambda i: (0, 0)),
            pl.BlockSpec((1, 128), lambda i: (0, 0)),
        ],
        out_specs=[
            pl.BlockSpec((TB, 128), lambda i: (i, 0)),
            pl.BlockSpec((1, 2, 128), lambda i: (i, 0, 0)),
        ],
        out_shape=[jax.ShapeDtypeStruct((T, 128), BF),
                   jax.ShapeDtypeStruct((NT, 2, 128), F32)],
        compiler_params=cp(dimension_semantics=("parallel",)),
        name="head_l3",
    )(o2, ps2, g2[None], be2[None], w3.T.astype(BF), b3[None])

    vt = jnp.asarray(_VALID_CODON)
    out = pl.pallas_call(
        functools.partial(_head4_body, ntok=ntok),
        grid=(NT,),
        in_specs=[
            pl.BlockSpec((TB, 128), lambda i: (i, 0)),
            pl.BlockSpec((NT, 2, 128), lambda i: (0, 0, 0)),
            pl.BlockSpec((1, 128), lambda i: (0, 0)),
            pl.BlockSpec((1, 128), lambda i: (0, 0)),
            pl.BlockSpec((128, V), lambda i: (0, 0)),
            pl.BlockSpec((1, V), lambda i: (0, 0)),
            pl.BlockSpec((TCS, B), lambda i: (i, 0)),
            pl.BlockSpec((21, V), lambda i: (0, 0)),
        ],
        out_specs=pl.BlockSpec((TB, V), lambda i: (i, 0)),
        out_shape=jax.ShapeDtypeStruct((T, V), F32),
        compiler_params=cp(dimension_semantics=("parallel",)),
        name="head_l4",
    )(o3, ps3, g3[None], be3[None], w4.T.astype(BF), b4[None], xT, vt)

    return out.reshape(S, B, V).swapaxes(0, 1)
